# baseline (speedup 1.0000x reference)
import jax
import jax.numpy as jnp
from jax.experimental import pallas as pl
from jax.experimental.pallas import tpu as pltpu

IGNORE = -1
NL = 128


def _loss_kernel(tgt_ref, logits_ref, out_ref,
                 psum_acc, max_acc, tgt_acc, tot_acc):
    rb = pl.program_id(0)
    i = pl.program_id(1)
    nrb = pl.num_programs(0)
    nv = pl.num_programs(1)
    rows, bv = logits_ref.shape
    vocab = nv * bv
    ng = bv // NL

    t_rel = tgt_ref[...] - i * bv
    t_b = jnp.broadcast_to(t_rel, (rows, NL))
    lane = jax.lax.broadcasted_iota(jnp.int32, (rows, NL), 1)

    psum = jnp.zeros((rows, NL), jnp.float32)
    mx = jnp.full((rows, NL), -jnp.inf, jnp.float32)
    tl = jnp.zeros((rows, NL), jnp.float32)
    for g in range(ng):
        xg = logits_ref[:, g * NL:(g + 1) * NL]
        is_t = (lane + g * NL) == t_b
        psum = psum + jax.nn.sigmoid(xg)
        mx = jnp.maximum(mx, jnp.where(is_t, -jnp.inf, xg))
        tl = tl + jnp.where(is_t, xg, 0.0)

    @pl.when(i == 0)
    def _init():
        psum_acc[...] = psum
        max_acc[...] = mx
        tgt_acc[...] = tl

    @pl.when(i > 0)
    def _update():
        psum_acc[...] += psum
        max_acc[...] = jnp.maximum(max_acc[...], mx)
        tgt_acc[...] += tl

    @pl.when(i == nv - 1)
    def _row_block_done():
        ps = jnp.sum(psum_acc[...], axis=1, keepdims=True)
        mo = jnp.max(max_acc[...], axis=1, keepdims=True)
        tlr = jnp.sum(tgt_acc[...], axis=1, keepdims=True)
        mask = (tgt_ref[...] != IGNORE).astype(jnp.float32)
        loss_simplex = (ps - 1.0) ** 2 / vocab
        loss_margin = jax.nn.softplus(mo - tlr)
        p_target = jax.nn.sigmoid(tlr)
        loss_brier = (1.0 - p_target) ** 2
        per_row = (loss_simplex + loss_margin + loss_brier) * mask
        part = jnp.concatenate(
            [jnp.sum(per_row, axis=0, keepdims=True),
             jnp.sum(mask, axis=0, keepdims=True)], axis=1)

        @pl.when(rb == 0)
        def _first():
            tot_acc[...] = part

        @pl.when(rb > 0)
        def _rest():
            tot_acc[...] += part

        @pl.when(rb == nrb - 1)
        def _emit():
            total = tot_acc[0, 0]
            count = tot_acc[0, 1]
            out_ref[...] = jnp.where(
                count > 0.0,
                jnp.full((1, 1), total / jnp.maximum(count, 1.0)),
                jnp.zeros((1, 1), jnp.float32))


def kernel(logits, targets):
    vocab = logits.shape[-1]
    logits2 = logits.reshape(-1, vocab)
    rows = logits2.shape[0]
    tgt2 = targets.reshape(rows, 1).astype(jnp.int32)

    bv = 3200
    rblk = 64
    nv = vocab // bv
    nrb = rows // rblk
    assert nv * bv == vocab and nrb * rblk == rows

    out = pl.pallas_call(
        _loss_kernel,
        grid=(nrb, nv),
        in_specs=[
            pl.BlockSpec((rblk, 1), lambda rb, i: (rb, 0)),
            pl.BlockSpec((rblk, bv), lambda rb, i: (rb, i)),
        ],
        out_specs=pl.BlockSpec((1, 1), lambda rb, i: (0, 0)),
        out_shape=jax.ShapeDtypeStruct((1, 1), jnp.float32),
        scratch_shapes=[
            pltpu.VMEM((rblk, NL), jnp.float32),
            pltpu.VMEM((rblk, NL), jnp.float32),
            pltpu.VMEM((rblk, NL), jnp.float32),
            pltpu.VMEM((1, 2), jnp.float32),
        ],
        compiler_params=pltpu.CompilerParams(
            dimension_semantics=("arbitrary", "arbitrary"),
        ),
    )(tgt2, logits2)
    return out[0, 0]


# speedup vs baseline: 2.0913x; 2.0913x over previous
import jax
import jax.numpy as jnp
from jax.experimental import pallas as pl
from jax.experimental.pallas import tpu as pltpu

IGNORE = -1


def _loss_kernel(tgt_ref, logits_ref, out_ref, tsum_acc, max_acc, tgt_acc):
    i = pl.program_id(0)
    nsteps = pl.num_programs(0)
    rows, bv = logits_ref.shape
    vocab = nsteps * bv

    x = logits_ref[...]
    t_rel = tgt_ref[...] - i * bv
    lane = jax.lax.broadcasted_iota(jnp.int32, x.shape, 1)
    is_t = lane == t_rel

    th = jnp.tanh(0.5 * x)
    tsum = jnp.sum(th, axis=1, keepdims=True)
    max_other = jnp.max(jnp.where(is_t, -jnp.inf, x), axis=1, keepdims=True)
    tgt_logit = jnp.sum(jnp.where(is_t, x, 0.0), axis=1, keepdims=True)

    @pl.when(i == 0)
    def _init():
        tsum_acc[...] = tsum
        max_acc[...] = max_other
        tgt_acc[...] = tgt_logit

    @pl.when(i > 0)
    def _update():
        tsum_acc[...] += tsum
        max_acc[...] = jnp.maximum(max_acc[...], max_other)
        tgt_acc[...] += tgt_logit

    @pl.when(i == nsteps - 1)
    def _finish():
        ps = 0.5 * tsum_acc[...] + 0.5 * vocab
        mo = max_acc[...]
        tlr = tgt_acc[...]
        mask = (tgt_ref[...] != IGNORE).astype(jnp.float32)
        loss_simplex = (ps - 1.0) ** 2 / vocab
        loss_margin = jax.nn.softplus(mo - tlr)
        p_target = jax.nn.sigmoid(tlr)
        loss_brier = (1.0 - p_target) ** 2
        per_row = (loss_simplex + loss_margin + loss_brier) * mask
        total = jnp.sum(per_row, axis=(0, 1), keepdims=True)
        count = jnp.sum(mask, axis=(0, 1), keepdims=True)
        out_ref[...] = jnp.where(count > 0.0,
                                 total / jnp.maximum(count, 1.0),
                                 0.0)


def kernel(logits, targets):
    vocab = logits.shape[-1]
    logits2 = logits.reshape(-1, vocab)
    rows = logits2.shape[0]
    tgt2 = targets.reshape(rows, 1).astype(jnp.int32)

    bv = 3200
    nsteps = vocab // bv
    assert nsteps * bv == vocab

    out = pl.pallas_call(
        _loss_kernel,
        grid=(nsteps,),
        in_specs=[
            pl.BlockSpec((rows, 1), lambda i: (0, 0)),
            pl.BlockSpec((rows, bv), lambda i: (0, i)),
        ],
        out_specs=pl.BlockSpec((1, 1), lambda i: (0, 0)),
        out_shape=jax.ShapeDtypeStruct((1, 1), jnp.float32),
        scratch_shapes=[
            pltpu.VMEM((rows, 1), jnp.float32),
            pltpu.VMEM((rows, 1), jnp.float32),
            pltpu.VMEM((rows, 1), jnp.float32),
        ],
        compiler_params=pltpu.CompilerParams(
            dimension_semantics=("arbitrary",),
        ),
    )(tgt2, logits2)
    return out[0, 0]


# speedup vs baseline: 2.1425x; 1.0245x over previous
import jax
import jax.numpy as jnp
from jax.experimental import pallas as pl
from jax.experimental.pallas import tpu as pltpu

IGNORE = -1


def _loss_kernel(tgt_ref, logits_ref, out_ref, tsum_acc, max_acc, tgt_acc):
    i = pl.program_id(0)
    nsteps = pl.num_programs(0)
    rows, bv = logits_ref.shape
    vocab = nsteps * bv

    x = logits_ref[...]
    t_rel = tgt_ref[...] - i * bv
    lane = jax.lax.broadcasted_iota(jnp.int32, x.shape, 1)
    is_t = lane == t_rel

    th = jnp.tanh(0.5 * x)
    tsum = jnp.sum(th, axis=1, keepdims=True)
    max_other = jnp.max(jnp.where(is_t, -jnp.inf, x), axis=1, keepdims=True)
    tgt_logit = jnp.sum(jnp.where(is_t, x, 0.0), axis=1, keepdims=True)

    @pl.when(i == 0)
    def _init():
        tsum_acc[...] = tsum
        max_acc[...] = max_other
        tgt_acc[...] = tgt_logit

    @pl.when(i > 0)
    def _update():
        tsum_acc[...] += tsum
        max_acc[...] = jnp.maximum(max_acc[...], max_other)
        tgt_acc[...] += tgt_logit

    @pl.when(i == nsteps - 1)
    def _finish():
        ps = 0.5 * tsum_acc[...] + 0.5 * vocab
        mo = max_acc[...]
        tlr = tgt_acc[...]
        mask = (tgt_ref[...] != IGNORE).astype(jnp.float32)
        loss_simplex = (ps - 1.0) ** 2 / vocab
        loss_margin = jax.nn.softplus(mo - tlr)
        p_target = jax.nn.sigmoid(tlr)
        loss_brier = (1.0 - p_target) ** 2
        per_row = (loss_simplex + loss_margin + loss_brier) * mask
        total = jnp.sum(per_row, axis=(0, 1), keepdims=True)
        count = jnp.sum(mask, axis=(0, 1), keepdims=True)
        out_ref[...] = jnp.where(count > 0.0,
                                 total / jnp.maximum(count, 1.0),
                                 0.0)


def kernel(logits, targets):
    vocab = logits.shape[-1]
    logits2 = logits.reshape(-1, vocab)
    rows = logits2.shape[0]
    tgt2 = targets.reshape(rows, 1).astype(jnp.int32)

    bv = 6400
    nsteps = vocab // bv
    assert nsteps * bv == vocab

    out = pl.pallas_call(
        _loss_kernel,
        grid=(nsteps,),
        in_specs=[
            pl.BlockSpec((rows, 1), lambda i: (0, 0)),
            pl.BlockSpec((rows, bv), lambda i: (0, i)),
        ],
        out_specs=pl.BlockSpec((1, 1), lambda i: (0, 0)),
        out_shape=jax.ShapeDtypeStruct((1, 1), jnp.float32),
        scratch_shapes=[
            pltpu.VMEM((rows, 1), jnp.float32),
            pltpu.VMEM((rows, 1), jnp.float32),
            pltpu.VMEM((rows, 1), jnp.float32),
        ],
        compiler_params=pltpu.CompilerParams(
            dimension_semantics=("arbitrary",),
        ),
    )(tgt2, logits2)
    return out[0, 0]
